# Initial kernel scaffold; baseline (speedup 1.0000x reference)
import jax
import jax.numpy as jnp
from jax.experimental import pallas as pl


def kernel(X, edge_index, rbf, dist, Ws1, bs1, Ws2, bs2, Ws3, bs3, Wt0, Wt1, Wt2, Wt3, Wt4, Wt5, ln_g, ln_b, Wl, bl):
    raise NotImplementedError("write your pallas kernel here")


# speedup vs baseline: 3.0015x; 3.0015x over previous
import functools

import jax
import jax.numpy as jnp
from jax import lax
from jax.experimental import pallas as pl
from jax.experimental.pallas import tpu as pltpu

N = 10000
E = 160000
H = 128
NUM_RBF0 = 32
RBF_DIM = 2 * NUM_RBF0


def _silu(v):
    return v * jax.nn.sigmoid(v)


def _node_prep_kernel(x_ref, wlt_ref, lng_ref, lnb_ref, bl_ref,
                      wat_ref, wbt_ref, w0t_ref, w1t_ref, w2t_ref,
                      p_ref, q_ref, xn_ref, t9_ref):
    xs = x_ref[...]
    d0 = xs[:, 0, :]
    d4 = xs[:, 4, :]
    d8 = xs[:, 8, :]
    lam = (d0 + d4 + d8) * (1.0 / 3.0)
    a01 = 0.5 * (xs[:, 1, :] - xs[:, 3, :])
    a02 = 0.5 * (xs[:, 2, :] - xs[:, 6, :])
    a12 = 0.5 * (xs[:, 5, :] - xs[:, 7, :])
    s01 = 0.5 * (xs[:, 1, :] + xs[:, 3, :])
    s02 = 0.5 * (xs[:, 2, :] + xs[:, 6, :])
    s12 = 0.5 * (xs[:, 5, :] + xs[:, 7, :])
    s00 = d0 - lam
    s11 = d4 - lam
    s22 = d8 - lam
    tI = 3.0 * lam * lam
    tA = 2.0 * (a01 * a01 + a02 * a02 + a12 * a12)
    tS = (s00 * s00 + s11 * s11 + s22 * s22
          + 2.0 * (s01 * s01 + s02 * s02 + s12 * s12))
    xc = jnp.concatenate([tI, tA, tS], axis=-1)
    mu = jnp.mean(xc, axis=-1, keepdims=True)
    ctr = xc - mu
    var = jnp.mean(ctr * ctr, axis=-1, keepdims=True)
    xn = ctr * lax.rsqrt(var + 1e-5) * lng_ref[...] + lnb_ref[...]
    h1 = jnp.dot(xn, wlt_ref[...], preferred_element_type=jnp.float32)
    h1 = h1 + bl_ref[...]
    xfeat = _silu(h1)
    p_ref[...] = jnp.dot(xfeat, wat_ref[...], preferred_element_type=jnp.float32)
    q_ref[...] = jnp.dot(xfeat, wbt_ref[...], preferred_element_type=jnp.float32)
    tX = jnp.sum(xs * xs, axis=1)
    inv = 1.0 / (tX + 1.0)
    xn_ref[...] = xs * inv[:, None, :]
    w0t = w0t_ref[...]
    w1t = w1t_ref[...]
    w2t = w2t_ref[...]
    t9_ref[:, 0, :] = jnp.dot(lam * inv, w0t, preferred_element_type=jnp.float32)
    t9_ref[:, 1, :] = jnp.dot(a01 * inv, w1t, preferred_element_type=jnp.float32)
    t9_ref[:, 2, :] = jnp.dot(a02 * inv, w1t, preferred_element_type=jnp.float32)
    t9_ref[:, 3, :] = jnp.dot(a12 * inv, w1t, preferred_element_type=jnp.float32)
    t9_ref[:, 4, :] = jnp.dot(s00 * inv, w2t, preferred_element_type=jnp.float32)
    t9_ref[:, 5, :] = jnp.dot(s11 * inv, w2t, preferred_element_type=jnp.float32)
    t9_ref[:, 6, :] = jnp.dot(s01 * inv, w2t, preferred_element_type=jnp.float32)
    t9_ref[:, 7, :] = jnp.dot(s02 * inv, w2t, preferred_element_type=jnp.float32)
    t9_ref[:, 8, :] = jnp.dot(s12 * inv, w2t, preferred_element_type=jnp.float32)


def _node_prep(X9, WlT, ln_g, ln_b, bl, WaT, WbT, W0T, W1T, W2T, interpret=False):
    BN = 200
    grid = (N // BN,)
    full = lambda shape: pl.BlockSpec(shape, lambda i: (0,) * len(shape))
    return pl.pallas_call(
        _node_prep_kernel,
        grid=grid,
        in_specs=[
            pl.BlockSpec((BN, 9, H), lambda i: (i, 0, 0)),
            full((3 * H, H)), full((1, 3 * H)), full((1, 3 * H)), full((1, H)),
            full((H, H)), full((H, H)), full((H, H)), full((H, H)), full((H, H)),
        ],
        out_specs=[
            pl.BlockSpec((BN, H), lambda i: (i, 0)),
            pl.BlockSpec((BN, H), lambda i: (i, 0)),
            pl.BlockSpec((BN, 9, H), lambda i: (i, 0, 0)),
            pl.BlockSpec((BN, 9, H), lambda i: (i, 0, 0)),
        ],
        out_shape=[
            jax.ShapeDtypeStruct((N, H), jnp.float32),
            jax.ShapeDtypeStruct((N, H), jnp.float32),
            jax.ShapeDtypeStruct((N, 9, H), jnp.float32),
            jax.ShapeDtypeStruct((N, 9, H), jnp.float32),
        ],
        interpret=interpret,
    )(X9, WlT, ln_g, ln_b, bl, WaT, WbT, W0T, W1T, W2T)


def _edge_mlp_kernel(g_ref, rbf_ref, wrt_ref, bs1_ref, w2t_ref, bs2_ref,
                     w3t_ref, bs3_ref, out_ref):
    r = jnp.dot(rbf_ref[...], wrt_ref[...], preferred_element_type=jnp.float32)
    e1 = _silu(g_ref[...] + r + bs1_ref[...])
    e2 = _silu(jnp.dot(e1, w2t_ref[...], preferred_element_type=jnp.float32)
               + bs2_ref[...])
    e3 = _silu(jnp.dot(e2, w3t_ref[...], preferred_element_type=jnp.float32)
               + bs3_ref[...])
    out_ref[...] = e3


def _edge_mlp(G, rbf, WrT, bs1, Ws2T, bs2, Ws3T, bs3, interpret=False):
    BE = 1000
    grid = (E // BE,)
    full = lambda shape: pl.BlockSpec(shape, lambda i: (0,) * len(shape))
    return pl.pallas_call(
        _edge_mlp_kernel,
        grid=grid,
        in_specs=[
            pl.BlockSpec((BE, H), lambda i: (i, 0)),
            pl.BlockSpec((BE, RBF_DIM), lambda i: (i, 0)),
            full((RBF_DIM, H)), full((1, H)),
            full((H, 2 * H)), full((1, 2 * H)),
            full((2 * H, 3 * H)), full((1, 3 * H)),
        ],
        out_specs=pl.BlockSpec((BE, 3 * H), lambda i: (i, 0)),
        out_shape=jax.ShapeDtypeStruct((E, 3 * H), jnp.float32),
        interpret=interpret,
    )(G, rbf, WrT, bs1, Ws2T, bs2, Ws3T, bs3)


def _full_from_compact(c):
    lam, a01, a02, a12, s00, s11, s01, s02, s12 = c
    s22 = -(s00 + s11)
    return [[lam + s00, s01 + a01, s02 + a02],
            [s01 - a01, lam + s11, s12 + a12],
            [s02 - a02, s12 - a12, lam + s22]]


def _mat3(Am, Bm):
    return [[sum(Am[a][k] * Bm[k][b] for k in range(3)) for b in range(3)]
            for a in range(3)]


def _final_kernel(m_ref, y_ref, xn_ref, w3t_ref, w4t_ref, w5t_ref, out_ref):
    mc = [m_ref[:, k, :] for k in range(9)]
    yc = [y_ref[:, k, :] for k in range(9)]
    M = _full_from_compact(mc)
    Y = _full_from_compact(yc)
    Ao = _mat3(M, Y)
    Bo = _mat3(Y, M)
    D = [[Ao[a][b] + Bo[a][b] for b in range(3)] for a in range(3)]
    lam = (D[0][0] + D[1][1] + D[2][2]) * (1.0 / 3.0)
    a01 = 0.5 * (D[0][1] - D[1][0])
    a02 = 0.5 * (D[0][2] - D[2][0])
    a12 = 0.5 * (D[1][2] - D[2][1])
    s00 = D[0][0] - lam
    s11 = D[1][1] - lam
    s01 = 0.5 * (D[0][1] + D[1][0])
    s02 = 0.5 * (D[0][2] + D[2][0])
    s12 = 0.5 * (D[1][2] + D[2][1])
    tn = sum(D[a][b] * D[a][b] for a in range(3) for b in range(3))
    inv = 1.0 / (tn + 1.0)
    w3t = w3t_ref[...]
    w4t = w4t_ref[...]
    w5t = w5t_ref[...]
    dot = lambda v, w: jnp.dot(v, w, preferred_element_type=jnp.float32)
    dc = [dot(lam * inv, w3t),
          dot(a01 * inv, w4t), dot(a02 * inv, w4t), dot(a12 * inv, w4t),
          dot(s00 * inv, w5t), dot(s11 * inv, w5t),
          dot(s01 * inv, w5t), dot(s02 * inv, w5t), dot(s12 * inv, w5t)]
    dX = _full_from_compact(dc)
    dX2 = _mat3(dX, dX)
    xnl = [xn_ref[:, k, :] for k in range(9)]
    for a in range(3):
        for b in range(3):
            out_ref[:, 3 * a + b, :] = xnl[3 * a + b] + dX[a][b] + dX2[a][b]


def _final(msg9, T9, Xn9, W3T, W4T, W5T, interpret=False):
    BN = 200
    grid = (N // BN,)
    full = lambda shape: pl.BlockSpec(shape, lambda i: (0,) * len(shape))
    return pl.pallas_call(
        _final_kernel,
        grid=grid,
        in_specs=[
            pl.BlockSpec((BN, 9, H), lambda i: (i, 0, 0)),
            pl.BlockSpec((BN, 9, H), lambda i: (i, 0, 0)),
            pl.BlockSpec((BN, 9, H), lambda i: (i, 0, 0)),
            full((H, H)), full((H, H)), full((H, H)),
        ],
        out_specs=pl.BlockSpec((BN, 9, H), lambda i: (i, 0, 0)),
        out_shape=jax.ShapeDtypeStruct((N, 9, H), jnp.float32),
        interpret=interpret,
    )(msg9, T9, Xn9, W3T, W4T, W5T)


def _edge_gather_jnp(P, Q, src, dst):
    return P[src] + Q[dst]


def _message_jnp(T9, ea_flat, src, dst):
    f = ea_flat.reshape(E, H, 3)
    t = T9[src]
    grp = jnp.array([0, 1, 1, 1, 2, 2, 2, 2, 2])
    fac = f[:, :, grp].transpose(0, 2, 1)
    contrib = t * fac
    msg = jax.ops.segment_sum(contrib, dst, num_segments=N)
    return msg


def kernel(X, edge_index, rbf, dist, Ws1, bs1, Ws2, bs2, Ws3, bs3,
           Wt0, Wt1, Wt2, Wt3, Wt4, Wt5, ln_g, ln_b, Wl, bl):
    src = edge_index[0]
    dst = edge_index[1]
    X9 = X.reshape(N, H, 9).transpose(0, 2, 1)
    WlT = Wl.T
    WaT = Ws1[:, :H].T
    WbT = Ws1[:, H:2 * H].T
    WrT = Ws1[:, 2 * H:].T
    Ws2T = Ws2.T
    Ws3T = Ws3.T
    r2 = lambda v: v.reshape(1, -1)

    P, Q, Xn9, T9 = _node_prep(X9, WlT, r2(ln_g), r2(ln_b), r2(bl),
                               WaT, WbT, Wt0.T, Wt1.T, Wt2.T)

    G = _edge_gather_jnp(P, Q, src, dst)
    ea_flat = _edge_mlp(G, rbf, WrT, r2(bs1), Ws2T, r2(bs2), Ws3T, r2(bs3))

    msg9 = _message_jnp(T9, ea_flat, src, dst)

    Xo9 = _final(msg9, T9, Xn9, Wt3.T, Wt4.T, Wt5.T)
    X_out = Xo9.transpose(0, 2, 1).reshape(N, H, 3, 3)
    edge_attr = ea_flat.reshape(E, H, 3)
    return (X_out, edge_attr)


# speedup vs baseline: 13.4151x; 4.4694x over previous
import functools

import jax
import jax.numpy as jnp
from jax import lax
from jax.experimental import pallas as pl
from jax.experimental.pallas import tpu as pltpu
from jax.experimental.pallas import tpu_sc as plsc

_NC = 2
_NS = 16
_L = 16
_NW = _NC * _NS

N = 10000
E = 160000
H = 128
NUM_RBF0 = 32
RBF_DIM = 2 * NUM_RBF0


def _silu(v):
    return v * jax.nn.sigmoid(v)


def _node_prep_kernel(x_ref, wlt_ref, lng_ref, lnb_ref, bl_ref,
                      wat_ref, wbt_ref, w0t_ref, w1t_ref, w2t_ref,
                      p_ref, q_ref, xn_ref, t9_ref):
    xs = x_ref[...]
    d0 = xs[:, 0, :]
    d4 = xs[:, 4, :]
    d8 = xs[:, 8, :]
    lam = (d0 + d4 + d8) * (1.0 / 3.0)
    a01 = 0.5 * (xs[:, 1, :] - xs[:, 3, :])
    a02 = 0.5 * (xs[:, 2, :] - xs[:, 6, :])
    a12 = 0.5 * (xs[:, 5, :] - xs[:, 7, :])
    s01 = 0.5 * (xs[:, 1, :] + xs[:, 3, :])
    s02 = 0.5 * (xs[:, 2, :] + xs[:, 6, :])
    s12 = 0.5 * (xs[:, 5, :] + xs[:, 7, :])
    s00 = d0 - lam
    s11 = d4 - lam
    s22 = d8 - lam
    tI = 3.0 * lam * lam
    tA = 2.0 * (a01 * a01 + a02 * a02 + a12 * a12)
    tS = (s00 * s00 + s11 * s11 + s22 * s22
          + 2.0 * (s01 * s01 + s02 * s02 + s12 * s12))
    xc = jnp.concatenate([tI, tA, tS], axis=-1)
    mu = jnp.mean(xc, axis=-1, keepdims=True)
    ctr = xc - mu
    var = jnp.mean(ctr * ctr, axis=-1, keepdims=True)
    xn = ctr * lax.rsqrt(var + 1e-5) * lng_ref[...] + lnb_ref[...]
    h1 = jnp.dot(xn, wlt_ref[...], preferred_element_type=jnp.float32)
    h1 = h1 + bl_ref[...]
    xfeat = _silu(h1)
    p_ref[...] = jnp.dot(xfeat, wat_ref[...], preferred_element_type=jnp.float32)
    q_ref[...] = jnp.dot(xfeat, wbt_ref[...], preferred_element_type=jnp.float32)
    tX = jnp.sum(xs * xs, axis=1)
    inv = 1.0 / (tX + 1.0)
    xn_ref[...] = xs * inv[:, None, :]
    w0t = w0t_ref[...]
    w1t = w1t_ref[...]
    w2t = w2t_ref[...]
    t9_ref[:, 0, :] = jnp.dot(lam * inv, w0t, preferred_element_type=jnp.float32)
    t9_ref[:, 1, :] = jnp.dot(a01 * inv, w1t, preferred_element_type=jnp.float32)
    t9_ref[:, 2, :] = jnp.dot(a02 * inv, w1t, preferred_element_type=jnp.float32)
    t9_ref[:, 3, :] = jnp.dot(a12 * inv, w1t, preferred_element_type=jnp.float32)
    t9_ref[:, 4, :] = jnp.dot(s00 * inv, w2t, preferred_element_type=jnp.float32)
    t9_ref[:, 5, :] = jnp.dot(s11 * inv, w2t, preferred_element_type=jnp.float32)
    t9_ref[:, 6, :] = jnp.dot(s01 * inv, w2t, preferred_element_type=jnp.float32)
    t9_ref[:, 7, :] = jnp.dot(s02 * inv, w2t, preferred_element_type=jnp.float32)
    t9_ref[:, 8, :] = jnp.dot(s12 * inv, w2t, preferred_element_type=jnp.float32)


def _node_prep(X9, WlT, ln_g, ln_b, bl, WaT, WbT, W0T, W1T, W2T, interpret=False):
    BN = 200
    grid = (N // BN,)
    full = lambda shape: pl.BlockSpec(shape, lambda i: (0,) * len(shape))
    return pl.pallas_call(
        _node_prep_kernel,
        grid=grid,
        in_specs=[
            pl.BlockSpec((BN, 9, H), lambda i: (i, 0, 0)),
            full((3 * H, H)), full((1, 3 * H)), full((1, 3 * H)), full((1, H)),
            full((H, H)), full((H, H)), full((H, H)), full((H, H)), full((H, H)),
        ],
        out_specs=[
            pl.BlockSpec((BN, H), lambda i: (i, 0)),
            pl.BlockSpec((BN, H), lambda i: (i, 0)),
            pl.BlockSpec((BN, 9, H), lambda i: (i, 0, 0)),
            pl.BlockSpec((BN, 9, H), lambda i: (i, 0, 0)),
        ],
        out_shape=[
            jax.ShapeDtypeStruct((N, H), jnp.float32),
            jax.ShapeDtypeStruct((N, H), jnp.float32),
            jax.ShapeDtypeStruct((N, 9, H), jnp.float32),
            jax.ShapeDtypeStruct((N, 9, H), jnp.float32),
        ],
        interpret=interpret,
    )(X9, WlT, ln_g, ln_b, bl, WaT, WbT, W0T, W1T, W2T)


def _edge_mlp_kernel(ps_ref, qd_ref, rbf_ref, wrt_ref, bs1_ref, w2t_ref, bs2_ref,
                     w3t_ref, bs3_ref, out_ref):
    r = jnp.dot(rbf_ref[...], wrt_ref[...], preferred_element_type=jnp.float32)
    e1 = _silu(ps_ref[...] + qd_ref[...] + r + bs1_ref[...])
    e2 = _silu(jnp.dot(e1, w2t_ref[...], preferred_element_type=jnp.float32)
               + bs2_ref[...])
    e3 = _silu(jnp.dot(e2, w3t_ref[...], preferred_element_type=jnp.float32)
               + bs3_ref[...])
    out_ref[...] = e3


def _edge_mlp(Ps, Qd, rbf, WrT, bs1, Ws2T, bs2, Ws3T, bs3, interpret=False):
    BE = 1000
    grid = (E // BE,)
    full = lambda shape: pl.BlockSpec(shape, lambda i: (0,) * len(shape))
    return pl.pallas_call(
        _edge_mlp_kernel,
        grid=grid,
        in_specs=[
            pl.BlockSpec((BE, H), lambda i: (i, 0)),
            pl.BlockSpec((BE, H), lambda i: (i, 0)),
            pl.BlockSpec((BE, RBF_DIM), lambda i: (i, 0)),
            full((RBF_DIM, H)), full((1, H)),
            full((H, 2 * H)), full((1, 2 * H)),
            full((2 * H, 3 * H)), full((1, 3 * H)),
        ],
        out_specs=pl.BlockSpec((BE, 3 * H), lambda i: (i, 0)),
        out_shape=jax.ShapeDtypeStruct((E, 3 * H), jnp.float32),
        interpret=interpret,
    )(Ps, Qd, rbf, WrT, bs1, Ws2T, bs2, Ws3T, bs3)


def _full_from_compact(c):
    lam, a01, a02, a12, s00, s11, s01, s02, s12 = c
    s22 = -(s00 + s11)
    return [[lam + s00, s01 + a01, s02 + a02],
            [s01 - a01, lam + s11, s12 + a12],
            [s02 - a02, s12 - a12, lam + s22]]


def _mat3(Am, Bm):
    return [[sum(Am[a][k] * Bm[k][b] for k in range(3)) for b in range(3)]
            for a in range(3)]


def _final_kernel(m_ref, y_ref, xn_ref, w3t_ref, w4t_ref, w5t_ref, out_ref):
    mc = [m_ref[:, k, :] for k in range(9)]
    yc = [y_ref[:, k, :] for k in range(9)]
    M = _full_from_compact(mc)
    Y = _full_from_compact(yc)
    Ao = _mat3(M, Y)
    Bo = _mat3(Y, M)
    D = [[Ao[a][b] + Bo[a][b] for b in range(3)] for a in range(3)]
    lam = (D[0][0] + D[1][1] + D[2][2]) * (1.0 / 3.0)
    a01 = 0.5 * (D[0][1] - D[1][0])
    a02 = 0.5 * (D[0][2] - D[2][0])
    a12 = 0.5 * (D[1][2] - D[2][1])
    s00 = D[0][0] - lam
    s11 = D[1][1] - lam
    s01 = 0.5 * (D[0][1] + D[1][0])
    s02 = 0.5 * (D[0][2] + D[2][0])
    s12 = 0.5 * (D[1][2] + D[2][1])
    tn = sum(D[a][b] * D[a][b] for a in range(3) for b in range(3))
    inv = 1.0 / (tn + 1.0)
    w3t = w3t_ref[...]
    w4t = w4t_ref[...]
    w5t = w5t_ref[...]
    dot = lambda v, w: jnp.dot(v, w, preferred_element_type=jnp.float32)
    dc = [dot(lam * inv, w3t),
          dot(a01 * inv, w4t), dot(a02 * inv, w4t), dot(a12 * inv, w4t),
          dot(s00 * inv, w5t), dot(s11 * inv, w5t),
          dot(s01 * inv, w5t), dot(s02 * inv, w5t), dot(s12 * inv, w5t)]
    dX = _full_from_compact(dc)
    dX2 = _mat3(dX, dX)
    xnl = [xn_ref[:, k, :] for k in range(9)]
    for a in range(3):
        for b in range(3):
            out_ref[:, 3 * a + b, :] = xnl[3 * a + b] + dX[a][b] + dX2[a][b]


def _final(msg9, T9, Xn9, W3T, W4T, W5T, interpret=False):
    BN = 200
    grid = (N // BN,)
    full = lambda shape: pl.BlockSpec(shape, lambda i: (0,) * len(shape))
    return pl.pallas_call(
        _final_kernel,
        grid=grid,
        in_specs=[
            pl.BlockSpec((BN, 9, H), lambda i: (i, 0, 0)),
            pl.BlockSpec((BN, 9, H), lambda i: (i, 0, 0)),
            pl.BlockSpec((BN, 9, H), lambda i: (i, 0, 0)),
            full((H, H)), full((H, H)), full((H, H)),
        ],
        out_specs=pl.BlockSpec((BN, 9, H), lambda i: (i, 0, 0)),
        out_shape=jax.ShapeDtypeStruct((N, 9, H), jnp.float32),
        interpret=interpret,
    )(msg9, T9, Xn9, W3T, W4T, W5T)


def _edge_gather_sc(P, Q, src, dst):
    per_w = E // _NW
    nfull = per_w // 128
    tail = per_w - nfull * 128
    mesh = plsc.VectorSubcoreMesh(core_axis_name="c", subcore_axis_name="s")

    @functools.partial(
        pl.kernel,
        out_type=[jax.ShapeDtypeStruct((E, H), jnp.float32),
                  jax.ShapeDtypeStruct((E, H), jnp.float32)],
        mesh=mesh,
        scratch_types=[
            pltpu.VMEM((128,), jnp.int32), pltpu.VMEM((128,), jnp.int32),
            pltpu.VMEM((tail,), jnp.int32), pltpu.VMEM((tail,), jnp.int32),
            pltpu.VMEM((128, H), jnp.float32), pltpu.VMEM((128, H), jnp.float32),
            pltpu.VMEM((tail, H), jnp.float32), pltpu.VMEM((tail, H), jnp.float32),
            pltpu.SemaphoreType.DMA, pltpu.SemaphoreType.DMA,
        ],
        compiler_params=pltpu.CompilerParams(use_tc_tiling_on_sc=False),
    )
    def k(p_hbm, q_hbm, src_hbm, dst_hbm, ps_hbm, qd_hbm,
          sidx, didx, sidx_t, didx_t, prow, qrow, prow_t, qrow_t, sem1, sem2):
        wid = lax.axis_index("s") * _NC + lax.axis_index("c")
        base_w = wid * per_w

        def do_block(base, si, di, pr, qr):
            B = pr.shape[0]
            pltpu.sync_copy(src_hbm.at[pl.ds(base, B)], si)
            pltpu.sync_copy(dst_hbm.at[pl.ds(base, B)], di)
            c1 = pltpu.async_copy(p_hbm.at[si], pr, sem1)
            c2 = pltpu.async_copy(q_hbm.at[di], qr, sem2)
            c1.wait()
            c2.wait()
            pltpu.sync_copy(pr, ps_hbm.at[pl.ds(base, B)])
            pltpu.sync_copy(qr, qd_hbm.at[pl.ds(base, B)])

        def body(i, _):
            do_block(base_w + i * 128, sidx, didx, prow, qrow)
            return 0

        lax.fori_loop(0, nfull, body, 0)
        do_block(base_w + nfull * 128, sidx_t, didx_t, prow_t, qrow_t)

    return k(P, Q, src, dst)


_GRP = (0, 1, 1, 1, 2, 2, 2, 2, 2)


_NPAD = 10240


def _message_sc(Tflat, FacFlat, src, dst):
    per_t = E // _NS
    nfull = per_t // 128
    tail = per_t - nfull * 128
    rows_t = _NPAD // _NS
    mesh = plsc.VectorSubcoreMesh(core_axis_name="c", subcore_axis_name="s")

    @functools.partial(
        pl.kernel,
        out_type=jax.ShapeDtypeStruct((8 * _NPAD, 144), jnp.float32),
        mesh=mesh,
        scratch_types=[
            pltpu.VMEM((128,), jnp.int32),
            pltpu.VMEM((128,), jnp.int32),
            pltpu.VMEM((tail,), jnp.int32),
            pltpu.VMEM((tail,), jnp.int32),
            pltpu.VMEM((128, 144), jnp.float32),
            pltpu.VMEM((128, 48), jnp.float32),
            pltpu.VMEM((tail, 144), jnp.float32),
            pltpu.VMEM((tail, 48), jnp.float32),
            pltpu.VMEM_SHARED((_NPAD, 144), jnp.float32),
            pltpu.SemaphoreType.DMA,
        ],
        compiler_params=pltpu.CompilerParams(use_tc_tiling_on_sc=False),
    )
    def k(t_hbm, f_hbm, src_hbm, dst_hbm, out_hbm,
          sidx, didx, sidx_t, didx_t, tbuf, fbuf, tbuf_t, fbuf_t, acc, sem):
        cid = lax.axis_index("c")
        sid = lax.axis_index("s")
        zv = jnp.zeros((_L,), jnp.float32)

        def fill_tbuf_zero():
            def zrow(r, _):
                for c in range(9):
                    tbuf[r, pl.ds(c * _L, _L)] = zv
                return 0
            lax.fori_loop(0, 128, zrow, 0)

        def zero_acc():
            for r5 in range(5):
                pltpu.sync_copy(tbuf, acc.at[pl.ds(sid * rows_t + r5 * 128, 128)])

        fill_tbuf_zero()
        zero_acc()

        def acc_block(base_e, chn, fbase, si, di, tb, fb):
            B = tb.shape[0]
            pltpu.sync_copy(src_hbm.at[pl.ds(base_e, B)], si)
            pltpu.sync_copy(dst_hbm.at[pl.ds(base_e, B)], di)

            def off(j, _):
                si[pl.ds(j * _L, _L)] = si[pl.ds(j * _L, _L)] + chn
                return 0
            lax.fori_loop(0, B // _L, off, 0)
            cp = pltpu.async_copy(t_hbm.at[si], tb, sem)
            pltpu.sync_copy(f_hbm.at[pl.ds(fbase + base_e, B)], fb)
            cp.wait()

            def edge(e, _):
                f3 = tuple(fb[e, pl.ds(g * _L, _L)] for g in range(3))
                for c in range(9):
                    tb[e, pl.ds(c * _L, _L)] = (
                        tb[e, pl.ds(c * _L, _L)] * f3[_GRP[c]])
                return 0
            lax.fori_loop(0, B, edge, 0)
            pltpu.sync_copy(tb, acc.at[di], add=True)

        for chi in range(4):
            chunk = cid * 4 + chi
            chn = chunk * N
            chp = chunk * _NPAD
            fbase = chunk * E
            plsc.subcore_barrier()

            def blk(i, _):
                acc_block(sid * per_t + i * 128, chn, fbase,
                          sidx, didx, tbuf, fbuf)
                return 0
            lax.fori_loop(0, nfull, blk, 0)
            acc_block(sid * per_t + nfull * 128, chn, fbase,
                      sidx_t, didx_t, tbuf_t, fbuf_t)

            plsc.subcore_barrier()
            for r5 in range(5):
                arow = sid * rows_t + r5 * 128
                pltpu.sync_copy(acc.at[pl.ds(arow, 128)], tbuf)
                pltpu.sync_copy(tbuf, out_hbm.at[pl.ds(chp + arow, 128)])
            fill_tbuf_zero()
            zero_acc()
            plsc.subcore_barrier()

    return k(Tflat, FacFlat, src, dst)


def kernel(X, edge_index, rbf, dist, Ws1, bs1, Ws2, bs2, Ws3, bs3,
           Wt0, Wt1, Wt2, Wt3, Wt4, Wt5, ln_g, ln_b, Wl, bl):
    src = edge_index[0]
    dst = edge_index[1]
    X9 = X.reshape(N, H, 9).transpose(0, 2, 1)
    WlT = Wl.T
    WaT = Ws1[:, :H].T
    WbT = Ws1[:, H:2 * H].T
    WrT = Ws1[:, 2 * H:].T
    Ws2T = Ws2.T
    Ws3T = Ws3.T
    r2 = lambda v: v.reshape(1, -1)

    P, Q, Xn9, T9 = _node_prep(X9, WlT, r2(ln_g), r2(ln_b), r2(bl),
                               WaT, WbT, Wt0.T, Wt1.T, Wt2.T)

    Ps, Qd = P[src], Q[dst]
    ea_flat = _edge_mlp(Ps, Qd, rbf, WrT, r2(bs1), Ws2T, r2(bs2), Ws3T, r2(bs3))

    Tflat = T9.reshape(N, 9, 8, _L).transpose(2, 0, 1, 3).reshape(8 * N, 144)
    FacFlat = (ea_flat.reshape(E, 8, _L, 3).transpose(1, 0, 3, 2)
               .reshape(8 * E, 48))
    msgf = _message_sc(Tflat, FacFlat, src, dst)
    msg9 = (msgf.reshape(8, _NPAD, 9, _L)[:, :N]
            .transpose(1, 2, 0, 3).reshape(N, 9, H))

    Xo9 = _final(msg9, T9, Xn9, Wt3.T, Wt4.T, Wt5.T)
    X_out = Xo9.transpose(0, 2, 1).reshape(N, H, 3, 3)
    edge_attr = ea_flat.reshape(E, H, 3)
    return (X_out, edge_attr)


# speedup vs baseline: 14.5131x; 1.0819x over previous
import functools

import jax
import jax.numpy as jnp
from jax import lax
from jax.experimental import pallas as pl
from jax.experimental.pallas import tpu as pltpu
from jax.experimental.pallas import tpu_sc as plsc

_NC = 2
_NS = 16
_L = 16
_NW = _NC * _NS

N = 10000
E = 160000
H = 128
NUM_RBF0 = 32
RBF_DIM = 2 * NUM_RBF0


def _silu(v):
    return v * jax.nn.sigmoid(v)


def _node_prep_kernel(x_ref, wlt_ref, lng_ref, lnb_ref, bl_ref,
                      wat_ref, wbt_ref, w0t_ref, w1t_ref, w2t_ref,
                      p_ref, q_ref, xn_ref, t9_ref):
    xs = x_ref[...]
    d0 = xs[:, 0, :]
    d4 = xs[:, 4, :]
    d8 = xs[:, 8, :]
    lam = (d0 + d4 + d8) * (1.0 / 3.0)
    a01 = 0.5 * (xs[:, 1, :] - xs[:, 3, :])
    a02 = 0.5 * (xs[:, 2, :] - xs[:, 6, :])
    a12 = 0.5 * (xs[:, 5, :] - xs[:, 7, :])
    s01 = 0.5 * (xs[:, 1, :] + xs[:, 3, :])
    s02 = 0.5 * (xs[:, 2, :] + xs[:, 6, :])
    s12 = 0.5 * (xs[:, 5, :] + xs[:, 7, :])
    s00 = d0 - lam
    s11 = d4 - lam
    s22 = d8 - lam
    tI = 3.0 * lam * lam
    tA = 2.0 * (a01 * a01 + a02 * a02 + a12 * a12)
    tS = (s00 * s00 + s11 * s11 + s22 * s22
          + 2.0 * (s01 * s01 + s02 * s02 + s12 * s12))
    xc = jnp.concatenate([tI, tA, tS], axis=-1)
    mu = jnp.mean(xc, axis=-1, keepdims=True)
    ctr = xc - mu
    var = jnp.mean(ctr * ctr, axis=-1, keepdims=True)
    xn = ctr * lax.rsqrt(var + 1e-5) * lng_ref[...] + lnb_ref[...]
    h1 = jnp.dot(xn, wlt_ref[...], preferred_element_type=jnp.float32)
    h1 = h1 + bl_ref[...]
    xfeat = _silu(h1)
    p_ref[...] = jnp.dot(xfeat, wat_ref[...], preferred_element_type=jnp.float32)
    q_ref[...] = jnp.dot(xfeat, wbt_ref[...], preferred_element_type=jnp.float32)
    tX = jnp.sum(xs * xs, axis=1)
    inv = 1.0 / (tX + 1.0)
    xn_ref[...] = xs * inv[:, None, :]
    w0t = w0t_ref[...]
    w1t = w1t_ref[...]
    w2t = w2t_ref[...]
    t9_ref[:, 0, :] = jnp.dot(lam * inv, w0t, preferred_element_type=jnp.float32)
    t9_ref[:, 1, :] = jnp.dot(a01 * inv, w1t, preferred_element_type=jnp.float32)
    t9_ref[:, 2, :] = jnp.dot(a02 * inv, w1t, preferred_element_type=jnp.float32)
    t9_ref[:, 3, :] = jnp.dot(a12 * inv, w1t, preferred_element_type=jnp.float32)
    t9_ref[:, 4, :] = jnp.dot(s00 * inv, w2t, preferred_element_type=jnp.float32)
    t9_ref[:, 5, :] = jnp.dot(s11 * inv, w2t, preferred_element_type=jnp.float32)
    t9_ref[:, 6, :] = jnp.dot(s01 * inv, w2t, preferred_element_type=jnp.float32)
    t9_ref[:, 7, :] = jnp.dot(s02 * inv, w2t, preferred_element_type=jnp.float32)
    t9_ref[:, 8, :] = jnp.dot(s12 * inv, w2t, preferred_element_type=jnp.float32)


def _node_prep(X9, WlT, ln_g, ln_b, bl, WaT, WbT, W0T, W1T, W2T, interpret=False):
    BN = 200
    grid = (N // BN,)
    full = lambda shape: pl.BlockSpec(shape, lambda i: (0,) * len(shape))
    return pl.pallas_call(
        _node_prep_kernel,
        grid=grid,
        in_specs=[
            pl.BlockSpec((BN, 9, H), lambda i: (i, 0, 0)),
            full((3 * H, H)), full((1, 3 * H)), full((1, 3 * H)), full((1, H)),
            full((H, H)), full((H, H)), full((H, H)), full((H, H)), full((H, H)),
        ],
        out_specs=[
            pl.BlockSpec((BN, H), lambda i: (i, 0)),
            pl.BlockSpec((BN, H), lambda i: (i, 0)),
            pl.BlockSpec((BN, 9, H), lambda i: (i, 0, 0)),
            pl.BlockSpec((BN, 9, H), lambda i: (i, 0, 0)),
        ],
        out_shape=[
            jax.ShapeDtypeStruct((N, H), jnp.float32),
            jax.ShapeDtypeStruct((N, H), jnp.float32),
            jax.ShapeDtypeStruct((N, 9, H), jnp.float32),
            jax.ShapeDtypeStruct((N, 9, H), jnp.float32),
        ],
        interpret=interpret,
    )(X9, WlT, ln_g, ln_b, bl, WaT, WbT, W0T, W1T, W2T)


def _edge_mlp_kernel(ps_ref, qd_ref, rbf_ref, wrt_ref, bs1_ref, w2t_ref, bs2_ref,
                     w3t_ref, bs3_ref, out_ref):
    r = jnp.dot(rbf_ref[...], wrt_ref[...], preferred_element_type=jnp.float32)
    e1 = _silu(ps_ref[...] + qd_ref[...] + r + bs1_ref[...])
    e2 = _silu(jnp.dot(e1, w2t_ref[...], preferred_element_type=jnp.float32)
               + bs2_ref[...])
    e3 = _silu(jnp.dot(e2, w3t_ref[...], preferred_element_type=jnp.float32)
               + bs3_ref[...])
    out_ref[...] = e3


def _edge_mlp(Ps, Qd, rbf, WrT, bs1, Ws2T, bs2, Ws3T, bs3, interpret=False):
    BE = 1000
    grid = (E // BE,)
    full = lambda shape: pl.BlockSpec(shape, lambda i: (0,) * len(shape))
    return pl.pallas_call(
        _edge_mlp_kernel,
        grid=grid,
        in_specs=[
            pl.BlockSpec((BE, H), lambda i: (i, 0)),
            pl.BlockSpec((BE, H), lambda i: (i, 0)),
            pl.BlockSpec((BE, RBF_DIM), lambda i: (i, 0)),
            full((RBF_DIM, H)), full((1, H)),
            full((H, 2 * H)), full((1, 2 * H)),
            full((2 * H, 3 * H)), full((1, 3 * H)),
        ],
        out_specs=pl.BlockSpec((BE, 3 * H), lambda i: (i, 0)),
        out_shape=jax.ShapeDtypeStruct((E, 3 * H), jnp.float32),
        interpret=interpret,
    )(Ps, Qd, rbf, WrT, bs1, Ws2T, bs2, Ws3T, bs3)


def _full_from_compact(c):
    lam, a01, a02, a12, s00, s11, s01, s02, s12 = c
    s22 = -(s00 + s11)
    return [[lam + s00, s01 + a01, s02 + a02],
            [s01 - a01, lam + s11, s12 + a12],
            [s02 - a02, s12 - a12, lam + s22]]


def _mat3(Am, Bm):
    return [[sum(Am[a][k] * Bm[k][b] for k in range(3)) for b in range(3)]
            for a in range(3)]


def _final_kernel(m_ref, y_ref, xn_ref, w3t_ref, w4t_ref, w5t_ref, out_ref):
    mc = [m_ref[:, k, :] for k in range(9)]
    yc = [y_ref[:, k, :] for k in range(9)]
    M = _full_from_compact(mc)
    Y = _full_from_compact(yc)
    Ao = _mat3(M, Y)
    Bo = _mat3(Y, M)
    D = [[Ao[a][b] + Bo[a][b] for b in range(3)] for a in range(3)]
    lam = (D[0][0] + D[1][1] + D[2][2]) * (1.0 / 3.0)
    a01 = 0.5 * (D[0][1] - D[1][0])
    a02 = 0.5 * (D[0][2] - D[2][0])
    a12 = 0.5 * (D[1][2] - D[2][1])
    s00 = D[0][0] - lam
    s11 = D[1][1] - lam
    s01 = 0.5 * (D[0][1] + D[1][0])
    s02 = 0.5 * (D[0][2] + D[2][0])
    s12 = 0.5 * (D[1][2] + D[2][1])
    tn = sum(D[a][b] * D[a][b] for a in range(3) for b in range(3))
    inv = 1.0 / (tn + 1.0)
    w3t = w3t_ref[...]
    w4t = w4t_ref[...]
    w5t = w5t_ref[...]
    dot = lambda v, w: jnp.dot(v, w, preferred_element_type=jnp.float32)
    dc = [dot(lam * inv, w3t),
          dot(a01 * inv, w4t), dot(a02 * inv, w4t), dot(a12 * inv, w4t),
          dot(s00 * inv, w5t), dot(s11 * inv, w5t),
          dot(s01 * inv, w5t), dot(s02 * inv, w5t), dot(s12 * inv, w5t)]
    dX = _full_from_compact(dc)
    dX2 = _mat3(dX, dX)
    xnl = [xn_ref[:, k, :] for k in range(9)]
    for a in range(3):
        for b in range(3):
            out_ref[:, 3 * a + b, :] = xnl[3 * a + b] + dX[a][b] + dX2[a][b]


def _final(msg9, T9, Xn9, W3T, W4T, W5T, interpret=False):
    BN = 200
    grid = (N // BN,)
    full = lambda shape: pl.BlockSpec(shape, lambda i: (0,) * len(shape))
    return pl.pallas_call(
        _final_kernel,
        grid=grid,
        in_specs=[
            pl.BlockSpec((BN, 9, H), lambda i: (i, 0, 0)),
            pl.BlockSpec((BN, 9, H), lambda i: (i, 0, 0)),
            pl.BlockSpec((BN, 9, H), lambda i: (i, 0, 0)),
            full((H, H)), full((H, H)), full((H, H)),
        ],
        out_specs=pl.BlockSpec((BN, 9, H), lambda i: (i, 0, 0)),
        out_shape=jax.ShapeDtypeStruct((N, 9, H), jnp.float32),
        interpret=interpret,
    )(msg9, T9, Xn9, W3T, W4T, W5T)


def _edge_gather_sc(P, Q, src, dst):
    per_w = E // _NW
    nfull = per_w // 128
    tail = per_w - nfull * 128
    mesh = plsc.VectorSubcoreMesh(core_axis_name="c", subcore_axis_name="s")

    @functools.partial(
        pl.kernel,
        out_type=[jax.ShapeDtypeStruct((E, H), jnp.float32),
                  jax.ShapeDtypeStruct((E, H), jnp.float32)],
        mesh=mesh,
        scratch_types=[
            pltpu.VMEM((128,), jnp.int32), pltpu.VMEM((128,), jnp.int32),
            pltpu.VMEM((tail,), jnp.int32), pltpu.VMEM((tail,), jnp.int32),
            pltpu.VMEM((128, H), jnp.float32), pltpu.VMEM((128, H), jnp.float32),
            pltpu.VMEM((tail, H), jnp.float32), pltpu.VMEM((tail, H), jnp.float32),
            pltpu.SemaphoreType.DMA, pltpu.SemaphoreType.DMA,
        ],
        compiler_params=pltpu.CompilerParams(use_tc_tiling_on_sc=False),
    )
    def k(p_hbm, q_hbm, src_hbm, dst_hbm, ps_hbm, qd_hbm,
          sidx, didx, sidx_t, didx_t, prow, qrow, prow_t, qrow_t, sem1, sem2):
        wid = lax.axis_index("s") * _NC + lax.axis_index("c")
        base_w = wid * per_w

        def do_block(base, si, di, pr, qr):
            B = pr.shape[0]
            pltpu.sync_copy(src_hbm.at[pl.ds(base, B)], si)
            pltpu.sync_copy(dst_hbm.at[pl.ds(base, B)], di)
            c1 = pltpu.async_copy(p_hbm.at[si], pr, sem1)
            c2 = pltpu.async_copy(q_hbm.at[di], qr, sem2)
            c1.wait()
            c2.wait()
            pltpu.sync_copy(pr, ps_hbm.at[pl.ds(base, B)])
            pltpu.sync_copy(qr, qd_hbm.at[pl.ds(base, B)])

        def body(i, _):
            do_block(base_w + i * 128, sidx, didx, prow, qrow)
            return 0

        lax.fori_loop(0, nfull, body, 0)
        do_block(base_w + nfull * 128, sidx_t, didx_t, prow_t, qrow_t)

    return k(P, Q, src, dst)


_GRP = (0, 1, 1, 1, 2, 2, 2, 2, 2)


_NPAD = 10240


def _message_sc(Tflat, FacFlat, src, dst):
    per_t = E // _NS
    nfull = per_t // 128
    tail = per_t - nfull * 128
    rows_t = _NPAD // _NS
    mesh = plsc.VectorSubcoreMesh(core_axis_name="c", subcore_axis_name="s")

    @functools.partial(
        pl.kernel,
        out_type=jax.ShapeDtypeStruct((8 * _NPAD, 144), jnp.float32),
        mesh=mesh,
        scratch_types=[
            pltpu.VMEM((128,), jnp.int32),
            pltpu.VMEM((128,), jnp.int32),
            pltpu.VMEM((tail,), jnp.int32),
            pltpu.VMEM((tail,), jnp.int32),
            pltpu.VMEM((128, 144), jnp.float32),
            pltpu.VMEM((128, 48), jnp.float32),
            pltpu.VMEM((tail, 144), jnp.float32),
            pltpu.VMEM((tail, 48), jnp.float32),
            pltpu.VMEM_SHARED((_NPAD, 144), jnp.float32),
            pltpu.SemaphoreType.DMA,
        ],
        compiler_params=pltpu.CompilerParams(use_tc_tiling_on_sc=False),
    )
    def k(t_hbm, f_hbm, src_hbm, dst_hbm, out_hbm,
          sidx, didx, sidx_t, didx_t, tbuf, fbuf, tbuf_t, fbuf_t, acc, sem):
        cid = lax.axis_index("c")
        sid = lax.axis_index("s")
        zv = jnp.zeros((_L,), jnp.float32)

        def fill_tbuf_zero():
            def zrow(r, _):
                for c in range(9):
                    tbuf[r, pl.ds(c * _L, _L)] = zv
                return 0
            lax.fori_loop(0, 128, zrow, 0)

        def zero_acc():
            for r5 in range(5):
                pltpu.sync_copy(tbuf, acc.at[pl.ds(sid * rows_t + r5 * 128, 128)])

        fill_tbuf_zero()
        zero_acc()

        def acc_block(base_e, chn, fbase, si, di, tb, fb):
            B = tb.shape[0]
            pltpu.sync_copy(src_hbm.at[pl.ds(base_e, B)], si)
            pltpu.sync_copy(dst_hbm.at[pl.ds(base_e, B)], di)

            def off(j, _):
                si[pl.ds(j * _L, _L)] = si[pl.ds(j * _L, _L)] + chn
                return 0
            lax.fori_loop(0, B // _L, off, 0)
            cp = pltpu.async_copy(t_hbm.at[si], tb, sem)
            pltpu.sync_copy(f_hbm.at[pl.ds(fbase + base_e, B)], fb)
            cp.wait()

            def edge(e, _):
                f3 = tuple(fb[e, pl.ds(g * _L, _L)] for g in range(3))
                for c in range(9):
                    tb[e, pl.ds(c * _L, _L)] = (
                        tb[e, pl.ds(c * _L, _L)] * f3[_GRP[c]])
                return 0
            lax.fori_loop(0, B, edge, 0)
            pltpu.sync_copy(tb, acc.at[di], add=True)

        for chi in range(4):
            chunk = cid * 4 + chi
            chn = chunk * N
            chp = chunk * _NPAD
            fbase = chunk * E
            plsc.subcore_barrier()

            def blk(i, _):
                acc_block(sid * per_t + i * 128, chn, fbase,
                          sidx, didx, tbuf, fbuf)
                return 0
            lax.fori_loop(0, nfull, blk, 0)
            acc_block(sid * per_t + nfull * 128, chn, fbase,
                      sidx_t, didx_t, tbuf_t, fbuf_t)

            plsc.subcore_barrier()
            for r5 in range(5):
                arow = sid * rows_t + r5 * 128
                pltpu.sync_copy(acc.at[pl.ds(arow, 128)], tbuf)
                pltpu.sync_copy(tbuf, out_hbm.at[pl.ds(chp + arow, 128)])
            fill_tbuf_zero()
            zero_acc()
            plsc.subcore_barrier()

    return k(Tflat, FacFlat, src, dst)


def kernel(X, edge_index, rbf, dist, Ws1, bs1, Ws2, bs2, Ws3, bs3,
           Wt0, Wt1, Wt2, Wt3, Wt4, Wt5, ln_g, ln_b, Wl, bl):
    src = edge_index[0]
    dst = edge_index[1]
    X9 = X.reshape(N, H, 9).transpose(0, 2, 1)
    WlT = Wl.T
    WaT = Ws1[:, :H].T
    WbT = Ws1[:, H:2 * H].T
    WrT = Ws1[:, 2 * H:].T
    Ws2T = Ws2.T
    Ws3T = Ws3.T
    r2 = lambda v: v.reshape(1, -1)

    P, Q, Xn9, T9 = _node_prep(X9, WlT, r2(ln_g), r2(ln_b), r2(bl),
                               WaT, WbT, Wt0.T, Wt1.T, Wt2.T)

    Ps, Qd = _edge_gather_sc(P, Q, src, dst)
    ea_flat = _edge_mlp(Ps, Qd, rbf, WrT, r2(bs1), Ws2T, r2(bs2), Ws3T, r2(bs3))

    Tflat = T9.reshape(N, 9, 8, _L).transpose(2, 0, 1, 3).reshape(8 * N, 144)
    FacFlat = (ea_flat.reshape(E, 8, _L, 3).transpose(1, 0, 3, 2)
               .reshape(8 * E, 48))
    msgf = _message_sc(Tflat, FacFlat, src, dst)
    msg9 = (msgf.reshape(8, _NPAD, 9, _L)[:, :N]
            .transpose(1, 2, 0, 3).reshape(N, 9, H))

    Xo9 = _final(msg9, T9, Xn9, Wt3.T, Wt4.T, Wt5.T)
    X_out = Xo9.transpose(0, 2, 1).reshape(N, H, 3, 3)
    edge_attr = ea_flat.reshape(E, H, 3)
    return (X_out, edge_attr)


# speedup vs baseline: 20.8373x; 1.4358x over previous
import functools

import jax
import jax.numpy as jnp
import numpy as np
from jax import lax
from jax.experimental import pallas as pl
from jax.experimental.pallas import tpu as pltpu
from jax.experimental.pallas import tpu_sc as plsc

_NC = 2
_NS = 16
_L = 16
_NW = _NC * _NS

N = 10000
E = 160000
H = 128
NUM_RBF0 = 32
RBF_DIM = 2 * NUM_RBF0


def _silu(v):
    return v * jax.nn.sigmoid(v)


def _build_p_in():
    p = np.zeros((9 * H, 9 * H), np.float32)
    hh, cc = np.meshgrid(np.arange(H), np.arange(9), indexing="ij")
    p[hh * 9 + cc, cc * H + hh] = 1.0
    return p


def _build_r():
    r = np.zeros((8, 144, 9 * H), np.float32)
    kk, ll = np.meshgrid(np.arange(9), np.arange(_L), indexing="ij")
    for ch in range(8):
        r[ch, kk * _L + ll, kk * H + ch * _L + ll] = 1.0
    return r


_P_IN = _build_p_in()
_P_OUT = _P_IN.T.copy()
_R_ASM = _build_r()
_PERM_F = np.array([( (j // 48) * _L + (j % _L)) * 3 + (j % 48) // _L
                    for j in range(384)], np.int32)


def _node_prep_kernel(x_ref, pin_ref, wlt_ref, lng_ref, lnb_ref, bl_ref,
                      wat_ref, wbt_ref, wke_ref,
                      p_ref, q_ref, xn_ref, tsc_ref):
    xp = jnp.dot(x_ref[...], pin_ref[...],
                 preferred_element_type=jnp.float32)
    d = [xp[:, c * H:(c + 1) * H] for c in range(9)]
    d0, d4, d8 = d[0], d[4], d[8]
    lam = (d0 + d4 + d8) * (1.0 / 3.0)
    a01 = 0.5 * (d[1] - d[3])
    a02 = 0.5 * (d[2] - d[6])
    a12 = 0.5 * (d[5] - d[7])
    s01 = 0.5 * (d[1] + d[3])
    s02 = 0.5 * (d[2] + d[6])
    s12 = 0.5 * (d[5] + d[7])
    s00 = d0 - lam
    s11 = d4 - lam
    s22 = d8 - lam
    tI = 3.0 * lam * lam
    tA = 2.0 * (a01 * a01 + a02 * a02 + a12 * a12)
    tS = (s00 * s00 + s11 * s11 + s22 * s22
          + 2.0 * (s01 * s01 + s02 * s02 + s12 * s12))
    xc = jnp.concatenate([tI, tA, tS], axis=-1)
    mu = jnp.mean(xc, axis=-1, keepdims=True)
    ctr = xc - mu
    var = jnp.mean(ctr * ctr, axis=-1, keepdims=True)
    xn = ctr * lax.rsqrt(var + 1e-5) * lng_ref[...] + lnb_ref[...]
    h1 = jnp.dot(xn, wlt_ref[...], preferred_element_type=jnp.float32)
    h1 = h1 + bl_ref[...]
    xfeat = _silu(h1)
    p_ref[...] = jnp.dot(xfeat, wat_ref[...], preferred_element_type=jnp.float32)
    q_ref[...] = jnp.dot(xfeat, wbt_ref[...], preferred_element_type=jnp.float32)
    tX = d[0] * d[0]
    for c in range(1, 9):
        tX = tX + d[c] * d[c]
    inv = 1.0 / (tX + 1.0)
    for c in range(9):
        xn_ref[:, c * H:(c + 1) * H] = d[c] * inv
    comp = (lam, a01, a02, a12, s00, s11, s01, s02, s12)
    wke = wke_ref[...]
    acc = jnp.dot(comp[0] * inv, wke[0], preferred_element_type=jnp.float32)
    for k in range(1, 9):
        acc = acc + jnp.dot(comp[k] * inv, wke[k],
                            preferred_element_type=jnp.float32)
    for ch in range(8):
        tsc_ref[ch] = acc[:, ch * 144:(ch + 1) * 144]


def _node_prep(Xf, Pin, WlT, ln_g, ln_b, bl, WaT, WbT, WkE, interpret=False):
    BN = 200
    grid = (N // BN,)
    full = lambda shape: pl.BlockSpec(shape, lambda i: (0,) * len(shape))
    return pl.pallas_call(
        _node_prep_kernel,
        grid=grid,
        in_specs=[
            pl.BlockSpec((BN, 9 * H), lambda i: (i, 0)),
            full((9 * H, 9 * H)),
            full((3 * H, H)), full((1, 3 * H)), full((1, 3 * H)), full((1, H)),
            full((H, H)), full((H, H)), full((9, H, 9 * H)),
        ],
        out_specs=[
            pl.BlockSpec((BN, H), lambda i: (i, 0)),
            pl.BlockSpec((BN, H), lambda i: (i, 0)),
            pl.BlockSpec((BN, 9 * H), lambda i: (i, 0)),
            pl.BlockSpec((8, BN, 144), lambda i: (0, i, 0)),
        ],
        out_shape=[
            jax.ShapeDtypeStruct((N, H), jnp.float32),
            jax.ShapeDtypeStruct((N, H), jnp.float32),
            jax.ShapeDtypeStruct((N, 9 * H), jnp.float32),
            jax.ShapeDtypeStruct((8, N, 144), jnp.float32),
        ],
        interpret=interpret,
    )(Xf, Pin, WlT, ln_g, ln_b, bl, WaT, WbT, WkE)


def _edge_mlp_kernel(ps_ref, qd_ref, rbf_ref, wrt_ref, bs1_ref, w2t_ref, bs2_ref,
                     w3t_ref, bs3_ref, w3tp_ref, bs3p_ref, out_ref, fsc_ref):
    r = jnp.dot(rbf_ref[...], wrt_ref[...], preferred_element_type=jnp.float32)
    e1 = _silu(ps_ref[...] + qd_ref[...] + r + bs1_ref[...])
    e2 = _silu(jnp.dot(e1, w2t_ref[...], preferred_element_type=jnp.float32)
               + bs2_ref[...])
    e3 = _silu(jnp.dot(e2, w3t_ref[...], preferred_element_type=jnp.float32)
               + bs3_ref[...])
    out_ref[...] = e3
    e3p = _silu(jnp.dot(e2, w3tp_ref[...], preferred_element_type=jnp.float32)
                + bs3p_ref[...])
    for ch in range(8):
        fsc_ref[ch] = e3p[:, ch * 48:(ch + 1) * 48]


def _edge_mlp(Ps, Qd, rbf, WrT, bs1, Ws2T, bs2, Ws3T, bs3, Ws3Tp, bs3p,
              interpret=False):
    BE = 1000
    grid = (E // BE,)
    full = lambda shape: pl.BlockSpec(shape, lambda i: (0,) * len(shape))
    return pl.pallas_call(
        _edge_mlp_kernel,
        grid=grid,
        in_specs=[
            pl.BlockSpec((BE, H), lambda i: (i, 0)),
            pl.BlockSpec((BE, H), lambda i: (i, 0)),
            pl.BlockSpec((BE, RBF_DIM), lambda i: (i, 0)),
            full((RBF_DIM, H)), full((1, H)),
            full((H, 2 * H)), full((1, 2 * H)),
            full((2 * H, 3 * H)), full((1, 3 * H)),
            full((2 * H, 3 * H)), full((1, 3 * H)),
        ],
        out_specs=[
            pl.BlockSpec((BE, 3 * H), lambda i: (i, 0)),
            pl.BlockSpec((8, BE, 48), lambda i: (0, i, 0)),
        ],
        out_shape=[
            jax.ShapeDtypeStruct((E, 3 * H), jnp.float32),
            jax.ShapeDtypeStruct((8, E, 48), jnp.float32),
        ],
        interpret=interpret,
    )(Ps, Qd, rbf, WrT, bs1, Ws2T, bs2, Ws3T, bs3, Ws3Tp, bs3p)


def _full_from_compact(c):
    lam, a01, a02, a12, s00, s11, s01, s02, s12 = c
    s22 = -(s00 + s11)
    return [[lam + s00, s01 + a01, s02 + a02],
            [s01 - a01, lam + s11, s12 + a12],
            [s02 - a02, s12 - a12, lam + s22]]


def _mat3(Am, Bm):
    return [[sum(Am[a][k] * Bm[k][b] for k in range(3)) for b in range(3)]
            for a in range(3)]


def _final_kernel(m_ref, y_ref, xn_ref, r_ref, pout_ref,
                  w3t_ref, w4t_ref, w5t_ref, out_ref):
    r_asm = r_ref[...]
    mcat = jnp.dot(m_ref[0], r_asm[0], preferred_element_type=jnp.float32)
    ycat = jnp.dot(y_ref[0], r_asm[0], preferred_element_type=jnp.float32)
    for ch in range(1, 8):
        mcat = mcat + jnp.dot(m_ref[ch], r_asm[ch],
                              preferred_element_type=jnp.float32)
        ycat = ycat + jnp.dot(y_ref[ch], r_asm[ch],
                              preferred_element_type=jnp.float32)
    mc = [mcat[:, k * H:(k + 1) * H] for k in range(9)]
    yc = [ycat[:, k * H:(k + 1) * H] for k in range(9)]
    M = _full_from_compact(mc)
    Y = _full_from_compact(yc)
    Ao = _mat3(M, Y)
    Bo = _mat3(Y, M)
    D = [[Ao[a][b] + Bo[a][b] for b in range(3)] for a in range(3)]
    lam = (D[0][0] + D[1][1] + D[2][2]) * (1.0 / 3.0)
    a01 = 0.5 * (D[0][1] - D[1][0])
    a02 = 0.5 * (D[0][2] - D[2][0])
    a12 = 0.5 * (D[1][2] - D[2][1])
    s00 = D[0][0] - lam
    s11 = D[1][1] - lam
    s01 = 0.5 * (D[0][1] + D[1][0])
    s02 = 0.5 * (D[0][2] + D[2][0])
    s12 = 0.5 * (D[1][2] + D[2][1])
    tn = sum(D[a][b] * D[a][b] for a in range(3) for b in range(3))
    inv = 1.0 / (tn + 1.0)
    w3t = w3t_ref[...]
    w4t = w4t_ref[...]
    w5t = w5t_ref[...]
    dot = lambda v, w: jnp.dot(v, w, preferred_element_type=jnp.float32)
    dc = [dot(lam * inv, w3t),
          dot(a01 * inv, w4t), dot(a02 * inv, w4t), dot(a12 * inv, w4t),
          dot(s00 * inv, w5t), dot(s11 * inv, w5t),
          dot(s01 * inv, w5t), dot(s02 * inv, w5t), dot(s12 * inv, w5t)]
    dX = _full_from_compact(dc)
    dX2 = _mat3(dX, dX)
    xnl = [xn_ref[:, c * H:(c + 1) * H] for c in range(9)]
    ocat = jnp.concatenate(
        [xnl[3 * a + b] + dX[a][b] + dX2[a][b]
         for a in range(3) for b in range(3)], axis=-1)
    out_ref[...] = jnp.dot(ocat, pout_ref[...],
                           preferred_element_type=jnp.float32)


def _final(msgc, Tsc, Xnf, Rasm, Pout, W3T, W4T, W5T, interpret=False):
    BN = 200
    grid = (N // BN,)
    full = lambda shape: pl.BlockSpec(shape, lambda i: (0,) * len(shape))
    return pl.pallas_call(
        _final_kernel,
        grid=grid,
        in_specs=[
            pl.BlockSpec((8, BN, 144), lambda i: (0, i, 0)),
            pl.BlockSpec((8, BN, 144), lambda i: (0, i, 0)),
            pl.BlockSpec((BN, 9 * H), lambda i: (i, 0)),
            full((8, 144, 9 * H)), full((9 * H, 9 * H)),
            full((H, H)), full((H, H)), full((H, H)),
        ],
        out_specs=pl.BlockSpec((BN, 9 * H), lambda i: (i, 0)),
        out_shape=jax.ShapeDtypeStruct((N, 9 * H), jnp.float32),
        interpret=interpret,
    )(msgc, Tsc, Xnf, Rasm, Pout, W3T, W4T, W5T)


def _edge_gather_sc(P, Q, src, dst):
    per_w = E // _NW
    nfull = per_w // 128
    tail = per_w - nfull * 128
    mesh = plsc.VectorSubcoreMesh(core_axis_name="c", subcore_axis_name="s")

    @functools.partial(
        pl.kernel,
        out_type=[jax.ShapeDtypeStruct((E, H), jnp.float32),
                  jax.ShapeDtypeStruct((E, H), jnp.float32)],
        mesh=mesh,
        scratch_types=[
            pltpu.VMEM((128,), jnp.int32), pltpu.VMEM((128,), jnp.int32),
            pltpu.VMEM((tail,), jnp.int32), pltpu.VMEM((tail,), jnp.int32),
            pltpu.VMEM((128, H), jnp.float32), pltpu.VMEM((128, H), jnp.float32),
            pltpu.VMEM((tail, H), jnp.float32), pltpu.VMEM((tail, H), jnp.float32),
            pltpu.SemaphoreType.DMA, pltpu.SemaphoreType.DMA,
        ],
        compiler_params=pltpu.CompilerParams(use_tc_tiling_on_sc=False),
    )
    def k(p_hbm, q_hbm, src_hbm, dst_hbm, ps_hbm, qd_hbm,
          sidx, didx, sidx_t, didx_t, prow, qrow, prow_t, qrow_t, sem1, sem2):
        wid = lax.axis_index("s") * _NC + lax.axis_index("c")
        base_w = wid * per_w

        def do_block(base, si, di, pr, qr):
            B = pr.shape[0]
            pltpu.sync_copy(src_hbm.at[pl.ds(base, B)], si)
            pltpu.sync_copy(dst_hbm.at[pl.ds(base, B)], di)
            c1 = pltpu.async_copy(p_hbm.at[si], pr, sem1)
            c2 = pltpu.async_copy(q_hbm.at[di], qr, sem2)
            c1.wait()
            c2.wait()
            pltpu.sync_copy(pr, ps_hbm.at[pl.ds(base, B)])
            pltpu.sync_copy(qr, qd_hbm.at[pl.ds(base, B)])

        def body(i, _):
            do_block(base_w + i * 128, sidx, didx, prow, qrow)
            return 0

        lax.fori_loop(0, nfull, body, 0)
        do_block(base_w + nfull * 128, sidx_t, didx_t, prow_t, qrow_t)

    return k(P, Q, src, dst)


_GRP = (0, 1, 1, 1, 2, 2, 2, 2, 2)


_NPAD = 10240


def _message_sc(Tflat, FacFlat, src, dst):
    per_t = E // _NS
    nfull = per_t // 128
    tail = per_t - nfull * 128
    rows_t = _NPAD // _NS
    mesh = plsc.VectorSubcoreMesh(core_axis_name="c", subcore_axis_name="s")

    @functools.partial(
        pl.kernel,
        out_type=jax.ShapeDtypeStruct((8 * _NPAD, 144), jnp.float32),
        mesh=mesh,
        scratch_types=[
            pltpu.VMEM((128,), jnp.int32),
            pltpu.VMEM((128,), jnp.int32),
            pltpu.VMEM((tail,), jnp.int32),
            pltpu.VMEM((tail,), jnp.int32),
            pltpu.VMEM((128, 144), jnp.float32),
            pltpu.VMEM((128, 48), jnp.float32),
            pltpu.VMEM((tail, 144), jnp.float32),
            pltpu.VMEM((tail, 48), jnp.float32),
            pltpu.VMEM_SHARED((_NPAD, 144), jnp.float32),
            pltpu.SemaphoreType.DMA,
        ],
        compiler_params=pltpu.CompilerParams(use_tc_tiling_on_sc=False),
    )
    def k(t_hbm, f_hbm, src_hbm, dst_hbm, out_hbm,
          sidx, didx, sidx_t, didx_t, tbuf, fbuf, tbuf_t, fbuf_t, acc, sem):
        cid = lax.axis_index("c")
        sid = lax.axis_index("s")
        zv = jnp.zeros((_L,), jnp.float32)

        def fill_tbuf_zero():
            def zrow(r, _):
                for c in range(9):
                    tbuf[r, pl.ds(c * _L, _L)] = zv
                return 0
            lax.fori_loop(0, 128, zrow, 0)

        def zero_acc():
            for r5 in range(5):
                pltpu.sync_copy(tbuf, acc.at[pl.ds(sid * rows_t + r5 * 128, 128)])

        fill_tbuf_zero()
        zero_acc()

        def acc_block(base_e, chn, fbase, si, di, tb, fb):
            B = tb.shape[0]
            pltpu.sync_copy(src_hbm.at[pl.ds(base_e, B)], si)
            pltpu.sync_copy(dst_hbm.at[pl.ds(base_e, B)], di)

            def off(j, _):
                si[pl.ds(j * _L, _L)] = si[pl.ds(j * _L, _L)] + chn
                return 0
            lax.fori_loop(0, B // _L, off, 0)
            cp = pltpu.async_copy(t_hbm.at[si], tb, sem)
            pltpu.sync_copy(f_hbm.at[pl.ds(fbase + base_e, B)], fb)
            cp.wait()

            def edge(e, _):
                f3 = tuple(fb[e, pl.ds(g * _L, _L)] for g in range(3))
                for c in range(9):
                    tb[e, pl.ds(c * _L, _L)] = (
                        tb[e, pl.ds(c * _L, _L)] * f3[_GRP[c]])
                return 0
            lax.fori_loop(0, B, edge, 0)
            pltpu.sync_copy(tb, acc.at[di], add=True)

        for chi in range(4):
            chunk = cid * 4 + chi
            chn = chunk * N
            chp = chunk * _NPAD
            fbase = chunk * E
            plsc.subcore_barrier()

            def blk(i, _):
                acc_block(sid * per_t + i * 128, chn, fbase,
                          sidx, didx, tbuf, fbuf)
                return 0
            lax.fori_loop(0, nfull, blk, 0)
            acc_block(sid * per_t + nfull * 128, chn, fbase,
                      sidx_t, didx_t, tbuf_t, fbuf_t)

            plsc.subcore_barrier()
            for r5 in range(5):
                arow = sid * rows_t + r5 * 128
                pltpu.sync_copy(acc.at[pl.ds(arow, 128)], tbuf)
                pltpu.sync_copy(tbuf, out_hbm.at[pl.ds(chp + arow, 128)])
            fill_tbuf_zero()
            zero_acc()
            plsc.subcore_barrier()

    return k(Tflat, FacFlat, src, dst)


def kernel(X, edge_index, rbf, dist, Ws1, bs1, Ws2, bs2, Ws3, bs3,
           Wt0, Wt1, Wt2, Wt3, Wt4, Wt5, ln_g, ln_b, Wl, bl):
    src = edge_index[0]
    dst = edge_index[1]
    Xf = X.reshape(N, 9 * H)
    WlT = Wl.T
    WaT = Ws1[:, :H].T
    WbT = Ws1[:, H:2 * H].T
    WrT = Ws1[:, 2 * H:].T
    Ws2T = Ws2.T
    Ws3T = Ws3.T
    r2 = lambda v: v.reshape(1, -1)

    wts = [Wt0.T] + [Wt1.T] * 3 + [Wt2.T] * 5
    WkE = jnp.stack([
        jnp.pad(w.reshape(H, 8, _L), ((0, 0), (0, 0), (k * _L, 144 - (k + 1) * _L))
                ).reshape(H, 9 * H)
        for k, w in enumerate(wts)])

    perm_f = jnp.asarray(_PERM_F)
    Ws3Tp = Ws3T[:, perm_f]
    bs3p = bs3[perm_f]

    P, Q, Xnf, Tsc = _node_prep(Xf, jnp.asarray(_P_IN), WlT, r2(ln_g),
                                r2(ln_b), r2(bl), WaT, WbT, WkE)

    Ps, Qd = _edge_gather_sc(P, Q, src, dst)
    ea_flat, fsc = _edge_mlp(Ps, Qd, rbf, WrT, r2(bs1), Ws2T, r2(bs2),
                             Ws3T, r2(bs3), Ws3Tp, r2(bs3p))

    Tflat = Tsc.reshape(8 * N, 144)
    FacFlat = fsc.reshape(8 * E, 48)
    msgf = _message_sc(Tflat, FacFlat, src, dst)

    Xo = _final(msgf.reshape(8, _NPAD, 144), Tsc, Xnf, jnp.asarray(_R_ASM),
                jnp.asarray(_P_OUT), Wt3.T, Wt4.T, Wt5.T)
    X_out = Xo.reshape(N, H, 3, 3)
    edge_attr = ea_flat.reshape(E, H, 3)
    return (X_out, edge_attr)


# speedup vs baseline: 22.8328x; 1.0958x over previous
import functools

import jax
import jax.numpy as jnp
import numpy as np
from jax import lax
from jax.experimental import pallas as pl
from jax.experimental.pallas import tpu as pltpu
from jax.experimental.pallas import tpu_sc as plsc

_NC = 2
_NS = 16
_L = 16
_NW = _NC * _NS

N = 10000
E = 160000
H = 128
NUM_RBF0 = 32
RBF_DIM = 2 * NUM_RBF0


def _silu(v):
    return v * jax.nn.sigmoid(v)


def _build_p_in():
    p = np.zeros((9 * H, 9 * H), np.float32)
    hh, cc = np.meshgrid(np.arange(H), np.arange(9), indexing="ij")
    p[hh * 9 + cc, cc * H + hh] = 1.0
    return p


def _build_r():
    r = np.zeros((8, 144, 9 * H), np.float32)
    kk, ll = np.meshgrid(np.arange(9), np.arange(_L), indexing="ij")
    for ch in range(8):
        r[ch, kk * _L + ll, kk * H + ch * _L + ll] = 1.0
    return r


_P_IN = _build_p_in()
_P_OUT = _P_IN.T.copy()
_R_ASM = _build_r()
_PERM_F = np.array([( (j // 48) * _L + (j % _L)) * 3 + (j % 48) // _L
                    for j in range(384)], np.int32)


def _node_prep_kernel(x_ref, pin_ref, wlt_ref, lng_ref, lnb_ref, bl_ref,
                      wat_ref, wbt_ref, wke_ref,
                      p_ref, q_ref, xn_ref, tsc_ref):
    xp = jnp.dot(x_ref[...], pin_ref[...],
                 preferred_element_type=jnp.float32)
    d = [xp[:, c * H:(c + 1) * H] for c in range(9)]
    d0, d4, d8 = d[0], d[4], d[8]
    lam = (d0 + d4 + d8) * (1.0 / 3.0)
    a01 = 0.5 * (d[1] - d[3])
    a02 = 0.5 * (d[2] - d[6])
    a12 = 0.5 * (d[5] - d[7])
    s01 = 0.5 * (d[1] + d[3])
    s02 = 0.5 * (d[2] + d[6])
    s12 = 0.5 * (d[5] + d[7])
    s00 = d0 - lam
    s11 = d4 - lam
    s22 = d8 - lam
    tI = 3.0 * lam * lam
    tA = 2.0 * (a01 * a01 + a02 * a02 + a12 * a12)
    tS = (s00 * s00 + s11 * s11 + s22 * s22
          + 2.0 * (s01 * s01 + s02 * s02 + s12 * s12))
    xc = jnp.concatenate([tI, tA, tS], axis=-1)
    mu = jnp.mean(xc, axis=-1, keepdims=True)
    ctr = xc - mu
    var = jnp.mean(ctr * ctr, axis=-1, keepdims=True)
    xn = ctr * lax.rsqrt(var + 1e-5) * lng_ref[...] + lnb_ref[...]
    h1 = jnp.dot(xn, wlt_ref[...], preferred_element_type=jnp.float32)
    h1 = h1 + bl_ref[...]
    xfeat = _silu(h1)
    p_ref[...] = jnp.dot(xfeat, wat_ref[...], preferred_element_type=jnp.float32)
    q_ref[...] = jnp.dot(xfeat, wbt_ref[...], preferred_element_type=jnp.float32)
    tX = d[0] * d[0]
    for c in range(1, 9):
        tX = tX + d[c] * d[c]
    inv = 1.0 / (tX + 1.0)
    for c in range(9):
        xn_ref[:, c * H:(c + 1) * H] = d[c] * inv
    comp = (lam, a01, a02, a12, s00, s11, s01, s02, s12)
    wke = wke_ref[...]
    acc = jnp.dot(comp[0] * inv, wke[0], preferred_element_type=jnp.float32)
    for k in range(1, 9):
        acc = acc + jnp.dot(comp[k] * inv, wke[k],
                            preferred_element_type=jnp.float32)
    for ch in range(8):
        tsc_ref[ch] = acc[:, ch * 144:(ch + 1) * 144]


def _node_prep(Xf, Pin, WlT, ln_g, ln_b, bl, WaT, WbT, WkE, interpret=False):
    BN = 200
    grid = (N // BN,)
    full = lambda shape: pl.BlockSpec(shape, lambda i: (0,) * len(shape))
    return pl.pallas_call(
        _node_prep_kernel,
        grid=grid,
        in_specs=[
            pl.BlockSpec((BN, 9 * H), lambda i: (i, 0)),
            full((9 * H, 9 * H)),
            full((3 * H, H)), full((1, 3 * H)), full((1, 3 * H)), full((1, H)),
            full((H, H)), full((H, H)), full((9, H, 9 * H)),
        ],
        out_specs=[
            pl.BlockSpec((BN, H), lambda i: (i, 0)),
            pl.BlockSpec((BN, H), lambda i: (i, 0)),
            pl.BlockSpec((BN, 9 * H), lambda i: (i, 0)),
            pl.BlockSpec((8, BN, 144), lambda i: (0, i, 0)),
        ],
        out_shape=[
            jax.ShapeDtypeStruct((N, H), jnp.float32),
            jax.ShapeDtypeStruct((N, H), jnp.float32),
            jax.ShapeDtypeStruct((N, 9 * H), jnp.float32),
            jax.ShapeDtypeStruct((8, N, 144), jnp.float32),
        ],
        interpret=interpret,
    )(Xf, Pin, WlT, ln_g, ln_b, bl, WaT, WbT, WkE)


def _edge_mlp_kernel(ps_ref, qd_ref, rbf_ref, wrt_ref, bs1_ref, w2t_ref, bs2_ref,
                     w3t_ref, bs3_ref, w3tp_ref, bs3p_ref, out_ref, fsc_ref):
    r = jnp.dot(rbf_ref[...], wrt_ref[...], preferred_element_type=jnp.float32)
    e1 = _silu(ps_ref[...] + qd_ref[...] + r + bs1_ref[...])
    e2 = _silu(jnp.dot(e1, w2t_ref[...], preferred_element_type=jnp.float32)
               + bs2_ref[...])
    e3 = _silu(jnp.dot(e2, w3t_ref[...], preferred_element_type=jnp.float32)
               + bs3_ref[...])
    out_ref[...] = e3
    e3p = _silu(jnp.dot(e2, w3tp_ref[...], preferred_element_type=jnp.float32)
                + bs3p_ref[...])
    for ch in range(8):
        fsc_ref[ch] = e3p[:, ch * 48:(ch + 1) * 48]


def _edge_mlp(Ps, Qd, rbf, WrT, bs1, Ws2T, bs2, Ws3T, bs3, Ws3Tp, bs3p,
              interpret=False):
    BE = 1000
    grid = (E // BE,)
    full = lambda shape: pl.BlockSpec(shape, lambda i: (0,) * len(shape))
    return pl.pallas_call(
        _edge_mlp_kernel,
        grid=grid,
        in_specs=[
            pl.BlockSpec((BE, H), lambda i: (i, 0)),
            pl.BlockSpec((BE, H), lambda i: (i, 0)),
            pl.BlockSpec((BE, RBF_DIM), lambda i: (i, 0)),
            full((RBF_DIM, H)), full((1, H)),
            full((H, 2 * H)), full((1, 2 * H)),
            full((2 * H, 3 * H)), full((1, 3 * H)),
            full((2 * H, 3 * H)), full((1, 3 * H)),
        ],
        out_specs=[
            pl.BlockSpec((BE, 3 * H), lambda i: (i, 0)),
            pl.BlockSpec((8, BE, 48), lambda i: (0, i, 0)),
        ],
        out_shape=[
            jax.ShapeDtypeStruct((E, 3 * H), jnp.float32),
            jax.ShapeDtypeStruct((8, E, 48), jnp.float32),
        ],
        interpret=interpret,
    )(Ps, Qd, rbf, WrT, bs1, Ws2T, bs2, Ws3T, bs3, Ws3Tp, bs3p)


def _full_from_compact(c):
    lam, a01, a02, a12, s00, s11, s01, s02, s12 = c
    s22 = -(s00 + s11)
    return [[lam + s00, s01 + a01, s02 + a02],
            [s01 - a01, lam + s11, s12 + a12],
            [s02 - a02, s12 - a12, lam + s22]]


def _mat3(Am, Bm):
    return [[sum(Am[a][k] * Bm[k][b] for k in range(3)) for b in range(3)]
            for a in range(3)]


def _final_kernel(m_ref, y_ref, xn_ref, r_ref, pout_ref,
                  w3t_ref, w4t_ref, w5t_ref, out_ref):
    r_asm = r_ref[...]
    mcat = jnp.dot(m_ref[0], r_asm[0], preferred_element_type=jnp.float32)
    ycat = jnp.dot(y_ref[0], r_asm[0], preferred_element_type=jnp.float32)
    for ch in range(1, 8):
        mcat = mcat + jnp.dot(m_ref[ch], r_asm[ch],
                              preferred_element_type=jnp.float32)
        ycat = ycat + jnp.dot(y_ref[ch], r_asm[ch],
                              preferred_element_type=jnp.float32)
    mc = [mcat[:, k * H:(k + 1) * H] for k in range(9)]
    yc = [ycat[:, k * H:(k + 1) * H] for k in range(9)]
    M = _full_from_compact(mc)
    Y = _full_from_compact(yc)
    Ao = _mat3(M, Y)
    Bo = _mat3(Y, M)
    D = [[Ao[a][b] + Bo[a][b] for b in range(3)] for a in range(3)]
    lam = (D[0][0] + D[1][1] + D[2][2]) * (1.0 / 3.0)
    a01 = 0.5 * (D[0][1] - D[1][0])
    a02 = 0.5 * (D[0][2] - D[2][0])
    a12 = 0.5 * (D[1][2] - D[2][1])
    s00 = D[0][0] - lam
    s11 = D[1][1] - lam
    s01 = 0.5 * (D[0][1] + D[1][0])
    s02 = 0.5 * (D[0][2] + D[2][0])
    s12 = 0.5 * (D[1][2] + D[2][1])
    tn = sum(D[a][b] * D[a][b] for a in range(3) for b in range(3))
    inv = 1.0 / (tn + 1.0)
    w3t = w3t_ref[...]
    w4t = w4t_ref[...]
    w5t = w5t_ref[...]
    dot = lambda v, w: jnp.dot(v, w, preferred_element_type=jnp.float32)
    dc = [dot(lam * inv, w3t),
          dot(a01 * inv, w4t), dot(a02 * inv, w4t), dot(a12 * inv, w4t),
          dot(s00 * inv, w5t), dot(s11 * inv, w5t),
          dot(s01 * inv, w5t), dot(s02 * inv, w5t), dot(s12 * inv, w5t)]
    dX = _full_from_compact(dc)
    dX2 = _mat3(dX, dX)
    xnl = [xn_ref[:, c * H:(c + 1) * H] for c in range(9)]
    ocat = jnp.concatenate(
        [xnl[3 * a + b] + dX[a][b] + dX2[a][b]
         for a in range(3) for b in range(3)], axis=-1)
    out_ref[...] = jnp.dot(ocat, pout_ref[...],
                           preferred_element_type=jnp.float32)


def _final(msgc, Tsc, Xnf, Rasm, Pout, W3T, W4T, W5T, interpret=False):
    BN = 200
    grid = (N // BN,)
    full = lambda shape: pl.BlockSpec(shape, lambda i: (0,) * len(shape))
    return pl.pallas_call(
        _final_kernel,
        grid=grid,
        in_specs=[
            pl.BlockSpec((8, BN, 144), lambda i: (0, i, 0)),
            pl.BlockSpec((8, BN, 144), lambda i: (0, i, 0)),
            pl.BlockSpec((BN, 9 * H), lambda i: (i, 0)),
            full((8, 144, 9 * H)), full((9 * H, 9 * H)),
            full((H, H)), full((H, H)), full((H, H)),
        ],
        out_specs=pl.BlockSpec((BN, 9 * H), lambda i: (i, 0)),
        out_shape=jax.ShapeDtypeStruct((N, 9 * H), jnp.float32),
        interpret=interpret,
    )(msgc, Tsc, Xnf, Rasm, Pout, W3T, W4T, W5T)


def _edge_gather_sc(P, Q, src, dst):
    per_w = E // _NW
    nfull = per_w // 128
    tail = per_w - nfull * 128
    mesh = plsc.VectorSubcoreMesh(core_axis_name="c", subcore_axis_name="s")

    @functools.partial(
        pl.kernel,
        out_type=[jax.ShapeDtypeStruct((E, H), jnp.float32),
                  jax.ShapeDtypeStruct((E, H), jnp.float32)],
        mesh=mesh,
        scratch_types=[
            pltpu.VMEM((128,), jnp.int32), pltpu.VMEM((128,), jnp.int32),
            pltpu.VMEM((tail,), jnp.int32), pltpu.VMEM((tail,), jnp.int32),
            pltpu.VMEM((128, H), jnp.float32), pltpu.VMEM((128, H), jnp.float32),
            pltpu.VMEM((tail, H), jnp.float32), pltpu.VMEM((tail, H), jnp.float32),
            pltpu.SemaphoreType.DMA, pltpu.SemaphoreType.DMA,
        ],
        compiler_params=pltpu.CompilerParams(use_tc_tiling_on_sc=False),
    )
    def k(p_hbm, q_hbm, src_hbm, dst_hbm, ps_hbm, qd_hbm,
          sidx, didx, sidx_t, didx_t, prow, qrow, prow_t, qrow_t, sem1, sem2):
        wid = lax.axis_index("s") * _NC + lax.axis_index("c")
        base_w = wid * per_w

        def do_block(base, si, di, pr, qr):
            B = pr.shape[0]
            pltpu.sync_copy(src_hbm.at[pl.ds(base, B)], si)
            pltpu.sync_copy(dst_hbm.at[pl.ds(base, B)], di)
            c1 = pltpu.async_copy(p_hbm.at[si], pr, sem1)
            c2 = pltpu.async_copy(q_hbm.at[di], qr, sem2)
            c1.wait()
            c2.wait()
            pltpu.sync_copy(pr, ps_hbm.at[pl.ds(base, B)])
            pltpu.sync_copy(qr, qd_hbm.at[pl.ds(base, B)])

        def body(i, _):
            do_block(base_w + i * 128, sidx, didx, prow, qrow)
            return 0

        lax.fori_loop(0, nfull, body, 0)
        do_block(base_w + nfull * 128, sidx_t, didx_t, prow_t, qrow_t)

    return k(P, Q, src, dst)


_GRP = (0, 1, 1, 1, 2, 2, 2, 2, 2)


_NPAD = 10240


def _message_sc(Tflat, FacFlat, src, dst):
    BK = 80
    nblk = E // _NS // BK
    npair = (nblk - 1) // 2
    rows_t = _NPAD // _NS
    mesh = plsc.VectorSubcoreMesh(core_axis_name="c", subcore_axis_name="s")

    @functools.partial(
        pl.kernel,
        out_type=jax.ShapeDtypeStruct((8 * _NPAD, 144), jnp.float32),
        mesh=mesh,
        scratch_types=[
            pltpu.VMEM((BK,), jnp.int32),
            pltpu.VMEM((BK,), jnp.int32),
            pltpu.VMEM((BK,), jnp.int32),
            pltpu.VMEM((BK,), jnp.int32),
            pltpu.VMEM((BK, 144), jnp.float32),
            pltpu.VMEM((BK, 48), jnp.float32),
            pltpu.VMEM((BK, 144), jnp.float32),
            pltpu.VMEM((BK, 48), jnp.float32),
            pltpu.VMEM_SHARED((_NPAD, 144), jnp.float32),
            pltpu.SemaphoreType.DMA,
            pltpu.SemaphoreType.DMA,
            pltpu.SemaphoreType.DMA,
            pltpu.SemaphoreType.DMA,
        ],
        compiler_params=pltpu.CompilerParams(use_tc_tiling_on_sc=False),
    )
    def k(t_hbm, f_hbm, src_hbm, dst_hbm, out_hbm,
          sidxa, didxa, sidxb, didxb,
          tbufa, fbufa, tbufb, fbufb, acc,
          semta, semfa, semtb, semfb):
        cid = lax.axis_index("c")
        sid = lax.axis_index("s")
        zv = jnp.zeros((_L,), jnp.float32)

        def fill_tbufa_zero():
            def zrow(r, _):
                for c in range(9):
                    tbufa[r, pl.ds(c * _L, _L)] = zv
                return 0
            lax.fori_loop(0, BK, zrow, 0)

        def zero_acc():
            for r8 in range(rows_t // BK):
                pltpu.sync_copy(tbufa, acc.at[pl.ds(sid * rows_t + r8 * BK, BK)])

        fill_tbufa_zero()
        zero_acc()

        def issue(base_e, chn, fbase, si, di, tb, fb, semt, semf):
            pltpu.sync_copy(src_hbm.at[pl.ds(base_e, BK)], si)
            pltpu.sync_copy(dst_hbm.at[pl.ds(base_e, BK)], di)

            def off(j, _):
                si[pl.ds(j * _L, _L)] = si[pl.ds(j * _L, _L)] + chn
                return 0
            lax.fori_loop(0, BK // _L, off, 0)
            pltpu.async_copy(t_hbm.at[si], tb, semt)
            pltpu.async_copy(f_hbm.at[pl.ds(fbase + base_e, BK)], fb, semf)

        def scale_scatter(si, di, tb, fb, semt, semf):
            pltpu.make_async_copy(t_hbm.at[si], tb, semt).wait()
            B = tb.shape[0]
            pltpu.make_async_copy(
                f_hbm.at[pl.ds(0, B)], fb, semf).wait()

            def edge(e, _):
                f3 = tuple(fb[e, pl.ds(g * _L, _L)] for g in range(3))
                for c in range(9):
                    tb[e, pl.ds(c * _L, _L)] = (
                        tb[e, pl.ds(c * _L, _L)] * f3[_GRP[c]])
                return 0
            lax.fori_loop(0, B, edge, 0)
            pltpu.sync_copy(tb, acc.at[di], add=True)

        for chi in range(4):
            chunk = cid * 4 + chi
            chn = chunk * N
            chp = chunk * _NPAD
            fbase = chunk * E
            base0 = sid * (E // _NS)
            plsc.subcore_barrier()

            issue(base0, chn, fbase, sidxa, didxa, tbufa, fbufa, semta, semfa)

            def pair(j, _):
                issue(base0 + (2 * j + 1) * BK, chn, fbase,
                      sidxb, didxb, tbufb, fbufb, semtb, semfb)
                scale_scatter(sidxa, didxa, tbufa, fbufa, semta, semfa)
                issue(base0 + (2 * j + 2) * BK, chn, fbase,
                      sidxa, didxa, tbufa, fbufa, semta, semfa)
                scale_scatter(sidxb, didxb, tbufb, fbufb, semtb, semfb)
                return 0

            lax.fori_loop(0, npair, pair, 0)
            scale_scatter(sidxa, didxa, tbufa, fbufa, semta, semfa)

            plsc.subcore_barrier()
            for r8 in range(rows_t // BK):
                arow = sid * rows_t + r8 * BK
                pltpu.sync_copy(acc.at[pl.ds(arow, BK)], tbufa)
                pltpu.sync_copy(tbufa, out_hbm.at[pl.ds(chp + arow, BK)])
            fill_tbufa_zero()
            zero_acc()
            plsc.subcore_barrier()

    return k(Tflat, FacFlat, src, dst)


def kernel(X, edge_index, rbf, dist, Ws1, bs1, Ws2, bs2, Ws3, bs3,
           Wt0, Wt1, Wt2, Wt3, Wt4, Wt5, ln_g, ln_b, Wl, bl):
    src = edge_index[0]
    dst = edge_index[1]
    Xf = X.reshape(N, 9 * H)
    WlT = Wl.T
    WaT = Ws1[:, :H].T
    WbT = Ws1[:, H:2 * H].T
    WrT = Ws1[:, 2 * H:].T
    Ws2T = Ws2.T
    Ws3T = Ws3.T
    r2 = lambda v: v.reshape(1, -1)

    wts = [Wt0.T] + [Wt1.T] * 3 + [Wt2.T] * 5
    WkE = jnp.stack([
        jnp.pad(w.reshape(H, 8, _L), ((0, 0), (0, 0), (k * _L, 144 - (k + 1) * _L))
                ).reshape(H, 9 * H)
        for k, w in enumerate(wts)])

    perm_f = jnp.asarray(_PERM_F)
    Ws3Tp = Ws3T[:, perm_f]
    bs3p = bs3[perm_f]

    P, Q, Xnf, Tsc = _node_prep(Xf, jnp.asarray(_P_IN), WlT, r2(ln_g),
                                r2(ln_b), r2(bl), WaT, WbT, WkE)

    Ps, Qd = _edge_gather_sc(P, Q, src, dst)
    ea_flat, fsc = _edge_mlp(Ps, Qd, rbf, WrT, r2(bs1), Ws2T, r2(bs2),
                             Ws3T, r2(bs3), Ws3Tp, r2(bs3p))

    Tflat = Tsc.reshape(8 * N, 144)
    FacFlat = fsc.reshape(8 * E, 48)
    msgf = _message_sc(Tflat, FacFlat, src, dst)

    Xo = _final(msgf.reshape(8, _NPAD, 144), Tsc, Xnf, jnp.asarray(_R_ASM),
                jnp.asarray(_P_OUT), Wt3.T, Wt4.T, Wt5.T)
    X_out = Xo.reshape(N, H, 3, 3)
    edge_attr = ea_flat.reshape(E, H, 3)
    return (X_out, edge_attr)
